# baseline (speedup 1.0000x reference)
import functools

import jax
import jax.numpy as jnp
from jax import lax
from jax.experimental import pallas as pl
from jax.experimental.pallas import tpu as pltpu
from jax.experimental.pallas import tpu_sc as plsc

N = 10000
IN = 128
H = 64
E = 160000

NC = 2
NS = 16
D1 = 128
D2 = 128
CHUNK = 96
EPW = E // NS
FULL = EPW // CHUNK
TAIL = EPW - FULL * CHUNK
NP = 10240
RPT = NP // NS

BR = 1000
GRID = N // BR


def _mm(a, w):
    return lax.dot_general(a, w, (((1,), (1,)), ((), ())),
                           preferred_element_type=jnp.float32,
                           precision=lax.Precision.HIGHEST)


def _proj_body(x_ref, wpl_ref, wpr_ref, bpr_ref, wnl_ref, wnr_ref, bnr_ref,
               tabp_ref, tabn_ref, basep_ref, basen_ref):
    x = x_ref[...]
    ones = jnp.ones((BR, 1), jnp.float32)
    zeros = jnp.zeros((BR, D1 - H - 1), jnp.float32)
    tabp_ref[...] = jnp.concatenate([_mm(x, wpl_ref[...]), ones, zeros], axis=1)
    tabn_ref[...] = jnp.concatenate([_mm(x, wnl_ref[...]), ones, zeros], axis=1)
    basep_ref[...] = _mm(x, wpr_ref[...]) + bpr_ref[...]
    basen_ref[...] = _mm(x, wnr_ref[...]) + bnr_ref[...]


def _tc_project(x, wpl, wpr, bpr, wnl, wnr, bnr):
    row = lambda i: (i, 0)
    full = lambda i: (0, 0)
    return pl.pallas_call(
        _proj_body,
        grid=(GRID,),
        in_specs=[
            pl.BlockSpec((BR, IN), row),
            pl.BlockSpec((H, IN), full), pl.BlockSpec((H, IN), full),
            pl.BlockSpec((1, H), full),
            pl.BlockSpec((H, IN), full), pl.BlockSpec((H, IN), full),
            pl.BlockSpec((1, H), full),
        ],
        out_specs=[
            pl.BlockSpec((BR, D1), row), pl.BlockSpec((BR, D1), row),
            pl.BlockSpec((BR, H), row), pl.BlockSpec((BR, H), row),
        ],
        out_shape=[
            jax.ShapeDtypeStruct((N, D1), jnp.float32),
            jax.ShapeDtypeStruct((N, D1), jnp.float32),
            jax.ShapeDtypeStruct((N, H), jnp.float32),
            jax.ShapeDtypeStruct((N, H), jnp.float32),
        ],
    )(x, wpl, wpr, bpr, wnl, wnr, bnr)


def _combine_body(sp_ref, sn_ref, basep_ref, basen_ref, h_ref):
    sp = sp_ref[...]
    sn = sn_ref[...]
    cp = jnp.maximum(sp[:, H:H + 1], 1.0)
    cn = jnp.maximum(sn[:, H:H + 1], 1.0)
    op = sp[:, :H] / cp + basep_ref[...]
    on = sn[:, :H] / cn + basen_ref[...]
    h_ref[...] = jnp.maximum(jnp.concatenate([op, on], axis=1), 0.0)


def _tc_combine(sp, sn, basep, basen):
    row = lambda i: (i, 0)
    return pl.pallas_call(
        _combine_body,
        grid=(GRID,),
        in_specs=[
            pl.BlockSpec((BR, D1), row), pl.BlockSpec((BR, D1), row),
            pl.BlockSpec((BR, H), row), pl.BlockSpec((BR, H), row),
        ],
        out_specs=pl.BlockSpec((BR, 2 * H), row),
        out_shape=jax.ShapeDtypeStruct((N, 2 * H), jnp.float32),
    )(sp, sn, basep, basen)


def _final_body(ap_ref, an_ref, h_ref, sp_ref, sn_ref,
                wpl2_ref, wpr2_ref, bpr2_ref, wnl2_ref, wnr2_ref, bnr2_ref,
                z_ref):
    cp = jnp.maximum(sp_ref[:, H:H + 1], 1.0)
    cn = jnp.maximum(sn_ref[:, H:H + 1], 1.0)
    mp = ap_ref[...] / cp
    mn = an_ref[...] / cn
    h = h_ref[...]
    pos_in = jnp.concatenate([mp[:, :H], mn[:, H:]], axis=1)
    neg_in = jnp.concatenate([mp[:, H:], mn[:, :H]], axis=1)
    op = _mm(pos_in, wpl2_ref[...]) + _mm(h[:, :H], wpr2_ref[...]) + bpr2_ref[...]
    on = _mm(neg_in, wnl2_ref[...]) + _mm(h[:, H:], wnr2_ref[...]) + bnr2_ref[...]
    z_ref[...] = jnp.maximum(jnp.concatenate([op, on], axis=1), 0.0)


def _tc_final(ap, an, h, sp, sn, wpl2, wpr2, bpr2, wnl2, wnr2, bnr2):
    row = lambda i: (i, 0)
    full = lambda i: (0, 0)
    return pl.pallas_call(
        _final_body,
        grid=(GRID,),
        in_specs=[
            pl.BlockSpec((BR, 2 * H), row), pl.BlockSpec((BR, 2 * H), row),
            pl.BlockSpec((BR, 2 * H), row),
            pl.BlockSpec((BR, D1), row), pl.BlockSpec((BR, D1), row),
            pl.BlockSpec((H, 2 * H), full), pl.BlockSpec((H, H), full),
            pl.BlockSpec((1, H), full),
            pl.BlockSpec((H, 2 * H), full), pl.BlockSpec((H, H), full),
            pl.BlockSpec((1, H), full),
        ],
        out_specs=pl.BlockSpec((BR, 2 * H), row),
        out_shape=jax.ShapeDtypeStruct((N, 2 * H), jnp.float32),
    )(ap, an, h, sp, sn, wpl2, wpr2, bpr2, wnl2, wnr2, bnr2)


def _make_agg(d, shared_table):
    mesh = plsc.VectorSubcoreMesh(core_axis_name="c", subcore_axis_name="s",
                                  num_cores=NC, num_subcores=NS)

    def body(*refs):
        if shared_table:
            (tab, zz, ps_r, pd_r, ns_r, nd_r, out,
             accum, sidx, didx, rows, sem) = refs
            tab_p = tab_n = tab
        else:
            (tab_p, tab_n, zz, ps_r, pd_r, ns_r, nd_r, out,
             accum, sidx, didx, rows, sem) = refs
        c = lax.axis_index("c")
        s = lax.axis_index("s")
        r0 = s * RPT

        def run(tab, src, dst, ci):
            pltpu.sync_copy(zz.at[pl.ds(r0, RPT)], accum.at[pl.ds(r0, RPT)])
            base = s * EPW
            pltpu.sync_copy(src.at[pl.ds(base, EPW)], sidx)
            pltpu.sync_copy(dst.at[pl.ds(base, EPW)], didx)
            plsc.subcore_barrier()

            def issue(k, j):
                pltpu.async_copy(tab.at[sidx.at[pl.ds(k * CHUNK, CHUNK)]],
                                 rows.at[j], sem)

            def gwait(j):
                pltpu.make_async_copy(tab.at[pl.ds(0, CHUNK)],
                                      rows.at[j], sem).wait()

            def scat(k, j):
                pltpu.sync_copy(rows.at[j],
                                accum.at[didx.at[pl.ds(k * CHUNK, CHUNK)]],
                                add=True)

            issue(0, 0)

            def pair(kk, carry):
                k0 = 2 * kk
                gwait(0)
                issue(k0 + 1, 1)
                scat(k0, 0)
                gwait(1)
                issue(k0 + 2, 0)
                scat(k0 + 1, 1)
                return carry

            lax.fori_loop(0, (FULL - 2) // 2, pair, 0)
            gwait(0)
            issue(FULL - 1, 1)
            scat(FULL - 2, 0)
            gwait(1)
            pltpu.async_copy(
                tab.at[sidx.at[pl.ds(FULL * CHUNK, TAIL)]],
                rows.at[0, pl.ds(0, TAIL)], sem)
            scat(FULL - 1, 1)
            pltpu.make_async_copy(tab.at[pl.ds(0, TAIL)],
                                  rows.at[0, pl.ds(0, TAIL)], sem).wait()
            pltpu.sync_copy(rows.at[0, pl.ds(0, TAIL)],
                            accum.at[didx.at[pl.ds(FULL * CHUNK, TAIL)]],
                            add=True)
            plsc.subcore_barrier()
            pltpu.sync_copy(accum.at[pl.ds(r0, RPT)],
                            out.at[ci, pl.ds(r0, RPT)])

        @pl.when(c == 0)
        def _():
            run(tab_p, ps_r, pd_r, 0)

        @pl.when(c == 1)
        def _():
            run(tab_n, ns_r, nd_r, 1)

    kern = pl.kernel(
        body,
        out_type=jax.ShapeDtypeStruct((2, NP, d), jnp.float32),
        mesh=mesh,
        scratch_types=[
            pltpu.VMEM_SHARED((NP, d), jnp.float32),
            pltpu.VMEM((EPW,), jnp.int32),
            pltpu.VMEM((EPW,), jnp.int32),
            pltpu.VMEM((2, CHUNK, d), jnp.float32),
            pltpu.SemaphoreType.DMA,
        ],
    )
    return kern


_agg1 = None
_agg2 = None


def kernel(x, pos_edge_index, neg_edge_index,
           W_pl1, W_pr1, b_pr1, W_nl1, W_nr1, b_nr1,
           W_pl2, W_pr2, b_pr2, W_nl2, W_nr2, b_nr2):
    global _agg1, _agg2
    if _agg1 is None:
        _agg1 = _make_agg(D1, shared_table=False)
        _agg2 = _make_agg(D2, shared_table=True)
    ps, pd = pos_edge_index[0], pos_edge_index[1]
    ns, nd = neg_edge_index[0], neg_edge_index[1]

    tabp, tabn, basep, basen = _tc_project(
        x, W_pl1, W_pr1, b_pr1.reshape(1, H), W_nl1, W_nr1, b_nr1.reshape(1, H))

    zz = jnp.zeros((NP, D1), jnp.float32)
    s1 = _agg1(tabp, tabn, zz, ps, pd, ns, nd)
    sp, sn = s1[0], s1[1]

    h = _tc_combine(sp, sn, basep, basen)

    s2 = _agg2(h, zz, ps, pd, ns, nd)

    z = _tc_final(s2[0], s2[1], h, sp, sn,
                  W_pl2, W_pr2, b_pr2.reshape(1, H),
                  W_nl2, W_nr2, b_nr2.reshape(1, H))
    return z


# speedup vs baseline: 1.1848x; 1.1848x over previous
import functools

import jax
import jax.numpy as jnp
from jax import lax
from jax.experimental import pallas as pl
from jax.experimental.pallas import tpu as pltpu
from jax.experimental.pallas import tpu_sc as plsc

N = 10000
IN = 128
H = 64
E = 160000

NC = 2
NS = 16
D1 = 128
D2 = 128
CHUNK = 96
EPW = E // NS
FULL = EPW // CHUNK
TAIL = EPW - FULL * CHUNK
NP = 10240
RPT = NP // NS

BR = 2000
GRID = N // BR


def _mm(a, w):
    return lax.dot_general(a, w, (((1,), (1,)), ((), ())),
                           preferred_element_type=jnp.float32)


def _proj_body(x_ref, wpl_ref, wpr_ref, bpr_ref, wnl_ref, wnr_ref, bnr_ref,
               tabp_ref, tabn_ref, basep_ref, basen_ref):
    x = x_ref[...]
    ones = jnp.ones((BR, 1), jnp.float32)
    zeros = jnp.zeros((BR, D1 - H - 1), jnp.float32)
    tabp_ref[...] = jnp.concatenate([_mm(x, wpl_ref[...]), ones, zeros], axis=1)
    tabn_ref[...] = jnp.concatenate([_mm(x, wnl_ref[...]), ones, zeros], axis=1)
    basep_ref[...] = _mm(x, wpr_ref[...]) + bpr_ref[...]
    basen_ref[...] = _mm(x, wnr_ref[...]) + bnr_ref[...]


def _tc_project(x, wpl, wpr, bpr, wnl, wnr, bnr):
    row = lambda i: (i, 0)
    full = lambda i: (0, 0)
    return pl.pallas_call(
        _proj_body,
        grid=(GRID,),
        in_specs=[
            pl.BlockSpec((BR, IN), row),
            pl.BlockSpec((H, IN), full), pl.BlockSpec((H, IN), full),
            pl.BlockSpec((1, H), full),
            pl.BlockSpec((H, IN), full), pl.BlockSpec((H, IN), full),
            pl.BlockSpec((1, H), full),
        ],
        out_specs=[
            pl.BlockSpec((BR, D1), row), pl.BlockSpec((BR, D1), row),
            pl.BlockSpec((BR, H), row), pl.BlockSpec((BR, H), row),
        ],
        out_shape=[
            jax.ShapeDtypeStruct((N, D1), jnp.float32),
            jax.ShapeDtypeStruct((N, D1), jnp.float32),
            jax.ShapeDtypeStruct((N, H), jnp.float32),
            jax.ShapeDtypeStruct((N, H), jnp.float32),
        ],
    )(x, wpl, wpr, bpr, wnl, wnr, bnr)


def _combine_body(sp_ref, sn_ref, basep_ref, basen_ref, h_ref):
    sp = sp_ref[0]
    sn = sn_ref[0]
    cp = jnp.maximum(sp[:, H:H + 1], 1.0)
    cn = jnp.maximum(sn[:, H:H + 1], 1.0)
    op = sp[:, :H] / cp + basep_ref[...]
    on = sn[:, :H] / cn + basen_ref[...]
    h_ref[...] = jnp.maximum(jnp.concatenate([op, on], axis=1), 0.0)


def _tc_combine(s1, basep, basen):
    row = lambda i: (i, 0)
    pos = lambda i: (0, i, 0)
    neg = lambda i: (1, i, 0)
    return pl.pallas_call(
        _combine_body,
        grid=(GRID,),
        in_specs=[
            pl.BlockSpec((1, BR, D1), pos), pl.BlockSpec((1, BR, D1), neg),
            pl.BlockSpec((BR, H), row), pl.BlockSpec((BR, H), row),
        ],
        out_specs=pl.BlockSpec((BR, 2 * H), row),
        out_shape=jax.ShapeDtypeStruct((N, 2 * H), jnp.float32),
    )(s1, s1, basep, basen)


def _final_body(ap_ref, an_ref, h_ref, sp_ref, sn_ref,
                wpl2_ref, wpr2_ref, bpr2_ref, wnl2_ref, wnr2_ref, bnr2_ref,
                z_ref):
    cp = jnp.maximum(sp_ref[0, :, H:H + 1], 1.0)
    cn = jnp.maximum(sn_ref[0, :, H:H + 1], 1.0)
    mp = ap_ref[0] / cp
    mn = an_ref[0] / cn
    h = h_ref[...]
    pos_in = jnp.concatenate([mp[:, :H], mn[:, H:]], axis=1)
    neg_in = jnp.concatenate([mp[:, H:], mn[:, :H]], axis=1)
    op = _mm(pos_in, wpl2_ref[...]) + _mm(h[:, :H], wpr2_ref[...]) + bpr2_ref[...]
    on = _mm(neg_in, wnl2_ref[...]) + _mm(h[:, H:], wnr2_ref[...]) + bnr2_ref[...]
    z_ref[...] = jnp.maximum(jnp.concatenate([op, on], axis=1), 0.0)


def _tc_final(s2, h, s1, wpl2, wpr2, bpr2, wnl2, wnr2, bnr2):
    row = lambda i: (i, 0)
    full = lambda i: (0, 0)
    pos = lambda i: (0, i, 0)
    neg = lambda i: (1, i, 0)
    return pl.pallas_call(
        _final_body,
        grid=(GRID,),
        in_specs=[
            pl.BlockSpec((1, BR, 2 * H), pos), pl.BlockSpec((1, BR, 2 * H), neg),
            pl.BlockSpec((BR, 2 * H), row),
            pl.BlockSpec((1, BR, D1), pos), pl.BlockSpec((1, BR, D1), neg),
            pl.BlockSpec((H, 2 * H), full), pl.BlockSpec((H, H), full),
            pl.BlockSpec((1, H), full),
            pl.BlockSpec((H, 2 * H), full), pl.BlockSpec((H, H), full),
            pl.BlockSpec((1, H), full),
        ],
        out_specs=pl.BlockSpec((BR, 2 * H), row),
        out_shape=jax.ShapeDtypeStruct((N, 2 * H), jnp.float32),
    )(s2, s2, h, s1, s1, wpl2, wpr2, bpr2, wnl2, wnr2, bnr2)


def _make_agg(d, shared_table):
    mesh = plsc.VectorSubcoreMesh(core_axis_name="c", subcore_axis_name="s",
                                  num_cores=NC, num_subcores=NS)

    def body(*refs):
        if shared_table:
            (tab, zz, ps_r, pd_r, ns_r, nd_r, out,
             accum, sidx, didx, rows, sem) = refs
            tab_p = tab_n = tab
        else:
            (tab_p, tab_n, zz, ps_r, pd_r, ns_r, nd_r, out,
             accum, sidx, didx, rows, sem) = refs
        c = lax.axis_index("c")
        s = lax.axis_index("s")
        r0 = s * RPT

        def run(tab, src, dst, ci):
            pltpu.sync_copy(zz.at[pl.ds(r0, RPT)], accum.at[pl.ds(r0, RPT)])
            base = s * EPW
            pltpu.sync_copy(src.at[pl.ds(base, EPW)], sidx)
            pltpu.sync_copy(dst.at[pl.ds(base, EPW)], didx)
            plsc.subcore_barrier()

            def issue(k, j):
                pltpu.async_copy(tab.at[sidx.at[pl.ds(k * CHUNK, CHUNK)]],
                                 rows.at[j], sem)

            def gwait(j):
                pltpu.make_async_copy(tab.at[pl.ds(0, CHUNK)],
                                      rows.at[j], sem).wait()

            def scat(k, j):
                pltpu.sync_copy(rows.at[j],
                                accum.at[didx.at[pl.ds(k * CHUNK, CHUNK)]],
                                add=True)

            issue(0, 0)

            def pair(kk, carry):
                k0 = 2 * kk
                gwait(0)
                issue(k0 + 1, 1)
                scat(k0, 0)
                gwait(1)
                issue(k0 + 2, 0)
                scat(k0 + 1, 1)
                return carry

            lax.fori_loop(0, (FULL - 2) // 2, pair, 0)
            gwait(0)
            issue(FULL - 1, 1)
            scat(FULL - 2, 0)
            gwait(1)
            pltpu.async_copy(
                tab.at[sidx.at[pl.ds(FULL * CHUNK, TAIL)]],
                rows.at[0, pl.ds(0, TAIL)], sem)
            scat(FULL - 1, 1)
            pltpu.make_async_copy(tab.at[pl.ds(0, TAIL)],
                                  rows.at[0, pl.ds(0, TAIL)], sem).wait()
            pltpu.sync_copy(rows.at[0, pl.ds(0, TAIL)],
                            accum.at[didx.at[pl.ds(FULL * CHUNK, TAIL)]],
                            add=True)
            plsc.subcore_barrier()
            pltpu.sync_copy(accum.at[pl.ds(r0, RPT)],
                            out.at[ci, pl.ds(r0, RPT)])

        @pl.when(c == 0)
        def _():
            run(tab_p, ps_r, pd_r, 0)

        @pl.when(c == 1)
        def _():
            run(tab_n, ns_r, nd_r, 1)

    kern = pl.kernel(
        body,
        out_type=jax.ShapeDtypeStruct((2, NP, d), jnp.float32),
        mesh=mesh,
        scratch_types=[
            pltpu.VMEM_SHARED((NP, d), jnp.float32),
            pltpu.VMEM((EPW,), jnp.int32),
            pltpu.VMEM((EPW,), jnp.int32),
            pltpu.VMEM((2, CHUNK, d), jnp.float32),
            pltpu.SemaphoreType.DMA,
        ],
    )
    return kern


_agg1 = None
_agg2 = None


def kernel(x, pos_edge_index, neg_edge_index,
           W_pl1, W_pr1, b_pr1, W_nl1, W_nr1, b_nr1,
           W_pl2, W_pr2, b_pr2, W_nl2, W_nr2, b_nr2):
    global _agg1, _agg2
    if _agg1 is None:
        _agg1 = _make_agg(D1, shared_table=False)
        _agg2 = _make_agg(D2, shared_table=True)
    ps, pd = pos_edge_index[0], pos_edge_index[1]
    ns, nd = neg_edge_index[0], neg_edge_index[1]

    tabp, tabn, basep, basen = _tc_project(
        x, W_pl1, W_pr1, b_pr1.reshape(1, H), W_nl1, W_nr1, b_nr1.reshape(1, H))

    zz = jnp.zeros((NP, D1), jnp.float32)
    s1 = _agg1(tabp, tabn, zz, ps, pd, ns, nd)

    h = _tc_combine(s1, basep, basen)

    s2 = _agg2(h, zz, ps, pd, ns, nd)

    z = _tc_final(s2, h, s1,
                  W_pl2, W_pr2, b_pr2.reshape(1, H),
                  W_nl2, W_nr2, b_nr2.reshape(1, H))
    return z


# speedup vs baseline: 1.2235x; 1.0327x over previous
import functools

import jax
import jax.numpy as jnp
from jax import lax
from jax.experimental import pallas as pl
from jax.experimental.pallas import tpu as pltpu
from jax.experimental.pallas import tpu_sc as plsc

N = 10000
IN = 128
H = 64
E = 160000

NC = 2
NS = 16
D1 = 128
D2 = 128
CHUNK = 96
EPW = E // NS
FULL = EPW // CHUNK
TAIL = EPW - FULL * CHUNK
NP = 10240
RPT = NP // NS

BR = 2000
GRID = N // BR


def _mm(a, w):
    return lax.dot_general(a, w, (((1,), (1,)), ((), ())),
                           preferred_element_type=jnp.float32)


def _proj_body(x_ref, wpl_ref, wpr_ref, bpr_ref, wnl_ref, wnr_ref, bnr_ref,
               tabp_ref, tabn_ref, basep_ref, basen_ref):
    x = x_ref[...]
    ones = jnp.ones((BR, 1), jnp.float32)
    zeros = jnp.zeros((BR, D1 - H - 1), jnp.float32)
    tabp_ref[...] = jnp.concatenate([_mm(x, wpl_ref[...]), ones, zeros], axis=1)
    tabn_ref[...] = jnp.concatenate([_mm(x, wnl_ref[...]), ones, zeros], axis=1)
    basep_ref[...] = _mm(x, wpr_ref[...]) + bpr_ref[...]
    basen_ref[...] = _mm(x, wnr_ref[...]) + bnr_ref[...]


def _tc_project(x, wpl, wpr, bpr, wnl, wnr, bnr):
    row = lambda i: (i, 0)
    full = lambda i: (0, 0)
    return pl.pallas_call(
        _proj_body,
        grid=(GRID,),
        in_specs=[
            pl.BlockSpec((BR, IN), row),
            pl.BlockSpec((H, IN), full), pl.BlockSpec((H, IN), full),
            pl.BlockSpec((1, H), full),
            pl.BlockSpec((H, IN), full), pl.BlockSpec((H, IN), full),
            pl.BlockSpec((1, H), full),
        ],
        out_specs=[
            pl.BlockSpec((BR, D1), row), pl.BlockSpec((BR, D1), row),
            pl.BlockSpec((BR, H), row), pl.BlockSpec((BR, H), row),
        ],
        out_shape=[
            jax.ShapeDtypeStruct((N, D1), jnp.float32),
            jax.ShapeDtypeStruct((N, D1), jnp.float32),
            jax.ShapeDtypeStruct((N, H), jnp.float32),
            jax.ShapeDtypeStruct((N, H), jnp.float32),
        ],
    )(x, wpl, wpr, bpr, wnl, wnr, bnr)


def _combine_body(sp_ref, sn_ref, basep_ref, basen_ref, h_ref):
    sp = sp_ref[0]
    sn = sn_ref[0]
    cp = jnp.maximum(sp[:, H:H + 1], 1.0)
    cn = jnp.maximum(sn[:, H:H + 1], 1.0)
    op = sp[:, :H] / cp + basep_ref[...]
    on = sn[:, :H] / cn + basen_ref[...]
    h_ref[...] = jnp.maximum(jnp.concatenate([op, on], axis=1), 0.0)


def _tc_combine(s1, basep, basen):
    row = lambda i: (i, 0)
    pos = lambda i: (0, i, 0)
    neg = lambda i: (1, i, 0)
    return pl.pallas_call(
        _combine_body,
        grid=(GRID,),
        in_specs=[
            pl.BlockSpec((1, BR, D1), pos), pl.BlockSpec((1, BR, D1), neg),
            pl.BlockSpec((BR, H), row), pl.BlockSpec((BR, H), row),
        ],
        out_specs=pl.BlockSpec((BR, 2 * H), row),
        out_shape=jax.ShapeDtypeStruct((N, 2 * H), jnp.float32),
    )(s1, s1, basep, basen)


def _final_body(ap_ref, an_ref, h_ref, sp_ref, sn_ref,
                wpl2_ref, wpr2_ref, bpr2_ref, wnl2_ref, wnr2_ref, bnr2_ref,
                z_ref):
    cp = jnp.maximum(sp_ref[0, :, H:H + 1], 1.0)
    cn = jnp.maximum(sn_ref[0, :, H:H + 1], 1.0)
    mp = ap_ref[0] / cp
    mn = an_ref[0] / cn
    h = h_ref[...]
    pos_in = jnp.concatenate([mp[:, :H], mn[:, H:]], axis=1)
    neg_in = jnp.concatenate([mp[:, H:], mn[:, :H]], axis=1)
    op = _mm(pos_in, wpl2_ref[...]) + _mm(h[:, :H], wpr2_ref[...]) + bpr2_ref[...]
    on = _mm(neg_in, wnl2_ref[...]) + _mm(h[:, H:], wnr2_ref[...]) + bnr2_ref[...]
    z_ref[...] = jnp.maximum(jnp.concatenate([op, on], axis=1), 0.0)


def _tc_final(s2, h, s1, wpl2, wpr2, bpr2, wnl2, wnr2, bnr2):
    row = lambda i: (i, 0)
    full = lambda i: (0, 0)
    pos = lambda i: (0, i, 0)
    neg = lambda i: (1, i, 0)
    return pl.pallas_call(
        _final_body,
        grid=(GRID,),
        in_specs=[
            pl.BlockSpec((1, BR, 2 * H), pos), pl.BlockSpec((1, BR, 2 * H), neg),
            pl.BlockSpec((BR, 2 * H), row),
            pl.BlockSpec((1, BR, D1), pos), pl.BlockSpec((1, BR, D1), neg),
            pl.BlockSpec((H, 2 * H), full), pl.BlockSpec((H, H), full),
            pl.BlockSpec((1, H), full),
            pl.BlockSpec((H, 2 * H), full), pl.BlockSpec((H, H), full),
            pl.BlockSpec((1, H), full),
        ],
        out_specs=pl.BlockSpec((BR, 2 * H), row),
        out_shape=jax.ShapeDtypeStruct((N, 2 * H), jnp.float32),
    )(s2, s2, h, s1, s1, wpl2, wpr2, bpr2, wnl2, wnr2, bnr2)


def _make_agg(d, shared_table):
    mesh = plsc.VectorSubcoreMesh(core_axis_name="c", subcore_axis_name="s",
                                  num_cores=NC, num_subcores=NS)

    def body(*refs):
        if shared_table:
            (tab, pe, ne, out, accum, sidx, didx, rows, sem) = refs
            tab_p = tab_n = tab
        else:
            (tab_p, tab_n, pe, ne, out, accum, sidx, didx, rows, sem) = refs
        c = lax.axis_index("c")
        s = lax.axis_index("s")
        r0 = s * RPT

        def run(tab, edges, ci):
            zv = jnp.zeros((16,), jnp.float32)

            def zrow(i, carry):
                for t in range(d // 16):
                    rows[0, i, pl.ds(t * 16, 16)] = zv
                return carry

            lax.fori_loop(0, CHUNK, zrow, 0)
            for i in range(RPT // CHUNK):
                pltpu.sync_copy(rows.at[0], accum.at[pl.ds(r0 + i * CHUNK,
                                                           CHUNK)])
            rem = RPT - (RPT // CHUNK) * CHUNK
            if rem:
                pltpu.sync_copy(rows.at[0, pl.ds(0, rem)],
                                accum.at[pl.ds(r0 + RPT - rem, rem)])
            base = s * EPW
            pltpu.sync_copy(edges.at[pl.ds(base, EPW)], sidx)
            pltpu.sync_copy(edges.at[pl.ds(E + base, EPW)], didx)
            plsc.subcore_barrier()

            def issue(k, j):
                pltpu.async_copy(tab.at[sidx.at[pl.ds(k * CHUNK, CHUNK)]],
                                 rows.at[j], sem)

            def gwait(j):
                pltpu.make_async_copy(tab.at[pl.ds(0, CHUNK)],
                                      rows.at[j], sem).wait()

            def scat(k, j):
                pltpu.sync_copy(rows.at[j],
                                accum.at[didx.at[pl.ds(k * CHUNK, CHUNK)]],
                                add=True)

            issue(0, 0)

            def pair(kk, carry):
                k0 = 2 * kk
                gwait(0)
                issue(k0 + 1, 1)
                scat(k0, 0)
                gwait(1)
                issue(k0 + 2, 0)
                scat(k0 + 1, 1)
                return carry

            lax.fori_loop(0, (FULL - 2) // 2, pair, 0)
            gwait(0)
            issue(FULL - 1, 1)
            scat(FULL - 2, 0)
            gwait(1)
            pltpu.async_copy(
                tab.at[sidx.at[pl.ds(FULL * CHUNK, TAIL)]],
                rows.at[0, pl.ds(0, TAIL)], sem)
            scat(FULL - 1, 1)
            pltpu.make_async_copy(tab.at[pl.ds(0, TAIL)],
                                  rows.at[0, pl.ds(0, TAIL)], sem).wait()
            pltpu.sync_copy(rows.at[0, pl.ds(0, TAIL)],
                            accum.at[didx.at[pl.ds(FULL * CHUNK, TAIL)]],
                            add=True)
            plsc.subcore_barrier()
            pltpu.sync_copy(accum.at[pl.ds(r0, RPT)],
                            out.at[ci, pl.ds(r0, RPT)])

        @pl.when(c == 0)
        def _():
            run(tab_p, pe, 0)

        @pl.when(c == 1)
        def _():
            run(tab_n, ne, 1)

    kern = pl.kernel(
        body,
        out_type=jax.ShapeDtypeStruct((2, NP, d), jnp.float32),
        mesh=mesh,
        scratch_types=[
            pltpu.VMEM_SHARED((NP, d), jnp.float32),
            pltpu.VMEM((EPW,), jnp.int32),
            pltpu.VMEM((EPW,), jnp.int32),
            pltpu.VMEM((2, CHUNK, d), jnp.float32),
            pltpu.SemaphoreType.DMA,
        ],
    )
    return kern


_agg1 = None
_agg2 = None


def kernel(x, pos_edge_index, neg_edge_index,
           W_pl1, W_pr1, b_pr1, W_nl1, W_nr1, b_nr1,
           W_pl2, W_pr2, b_pr2, W_nl2, W_nr2, b_nr2):
    global _agg1, _agg2
    if _agg1 is None:
        _agg1 = _make_agg(D1, shared_table=False)
        _agg2 = _make_agg(D2, shared_table=True)

    tabp, tabn, basep, basen = _tc_project(
        x, W_pl1, W_pr1, b_pr1.reshape(1, H), W_nl1, W_nr1, b_nr1.reshape(1, H))

    pe = pos_edge_index.reshape(-1)
    ne = neg_edge_index.reshape(-1)
    s1 = _agg1(tabp, tabn, pe, ne)

    h = _tc_combine(s1, basep, basen)

    s2 = _agg2(h, pe, ne)

    z = _tc_final(s2, h, s1,
                  W_pl2, W_pr2, b_pr2.reshape(1, H),
                  W_nl2, W_nr2, b_nr2.reshape(1, H))
    return z
